# baseline (speedup 1.0000x reference)
import functools

import jax
import jax.numpy as jnp
from jax import lax
from jax.experimental import pallas as pl
from jax.experimental.pallas import tpu as pltpu
from jax.experimental.pallas import tpu_sc as plsc

N_OBJ = 2048
NUM_CLS = 151
HIDDEN = 256
P = 131072
NN = N_OBJ * N_OBJ

NW = 32
PPW = P // NW
PROWS = PPW * 2 // 128
NCH = 4
CH = PPW // NCH


def _tc_vals_body(lg_ref, ws_ref, bs_ref, wo_ref, bo_ref, wcs_ref, wco_ref,
                  bc_ref, sval_ref, oval_ref):
    x = lg_ref[...]
    m = jnp.max(x, axis=1, keepdims=True)
    e = jnp.exp(x - m)
    p = e / jnp.sum(e, axis=1, keepdims=True)
    hs = jnp.maximum(
        jnp.dot(p, ws_ref[...], preferred_element_type=jnp.float32) + bs_ref[...], 0.0)
    ho = jnp.maximum(
        jnp.dot(p, wo_ref[...], preferred_element_type=jnp.float32) + bo_ref[...], 0.0)
    sval_ref[...] = jnp.sum(hs * wcs_ref[...], axis=1) + bc_ref[0, 0]
    oval_ref[...] = jnp.sum(ho * wco_ref[...], axis=1)


_tc_vals = pl.pallas_call(
    _tc_vals_body,
    out_shape=(jax.ShapeDtypeStruct((N_OBJ,), jnp.float32),
               jax.ShapeDtypeStruct((N_OBJ,), jnp.float32)),
)


def _sc_body(pairs_hbm, sval_hbm, oval_hbm, mat_hbm, logits_hbm,
             pair_v, stab_v, otab_v, logit_v, score_v, fidx_v,
             stsem, ssem, lsem):
    c = lax.axis_index("c")
    s = lax.axis_index("s")
    w = c * 16 + s

    d1 = pltpu.async_copy(
        pairs_hbm.at[pl.ds(w * PROWS, PROWS), :], pair_v, stsem)
    d2 = pltpu.async_copy(sval_hbm, stab_v, stsem)
    d3 = pltpu.async_copy(oval_hbm, otab_v, stsem)
    d1.wait()
    d2.wait()
    d3.wait()

    zero16 = jnp.zeros((16,), jnp.int32)

    def body(m, carry):
        lane = lax.iota(jnp.int32, 16)
        flat = m * 32 + 2 * lane
        ii = plsc.load_gather(pair_v, [flat // 128, flat % 128])
        jj = plsc.load_gather(pair_v, [(flat + 1) // 128, (flat + 1) % 128])
        sv = plsc.load_gather(stab_v, [ii])
        ov = plsc.load_gather(otab_v, [jj])
        lg = sv + ov
        logit_v[pl.ds(m * 16, 16)] = lg
        sc = 1.0 / (1.0 + jnp.exp(-lg))
        score_v[pl.ds(m * 16, 16)] = sc
        fidx_v[pl.ds(m * 16, 16)] = ii * N_OBJ + jj
        return carry
    lax.fori_loop(0, PPW // 16, body, 0)

    ds = pltpu.async_copy(score_v, mat_hbm.at[fidx_v], ssem)
    dl = pltpu.async_copy(logit_v, logits_hbm.at[pl.ds(w * PPW, PPW)], lsem)
    ds.wait()
    dl.wait()


_sc_scatter = functools.partial(
    pl.kernel,
    out_type=jax.ShapeDtypeStruct((P,), jnp.float32),
    mesh=plsc.VectorSubcoreMesh(core_axis_name="c", subcore_axis_name="s"),
    compiler_params=pltpu.CompilerParams(
        needs_layout_passes=False,
        disable_bounds_checks=True,
        disable_semaphore_checks=True,
        skip_device_barrier=True,
    ),
    scratch_types=(
        pltpu.VMEM((PROWS, 128), jnp.int32),
        pltpu.VMEM((N_OBJ,), jnp.float32),
        pltpu.VMEM((N_OBJ,), jnp.float32),
        pltpu.VMEM((PPW,), jnp.float32),
        pltpu.VMEM((PPW,), jnp.float32),
        pltpu.VMEM((PPW,), jnp.int32),
        pltpu.SemaphoreType.DMA,
        pltpu.SemaphoreType.DMA,
        pltpu.SemaphoreType.DMA,
    ),
)(_sc_body)


def kernel(visual_feat, pred_logits, pair_idx, W_sub, b_sub, W_obj, b_obj,
           W_cls, b_cls):
    del visual_feat
    ws_cls = W_cls[:HIDDEN].reshape(1, HIDDEN)
    wo_cls = W_cls[HIDDEN:].reshape(1, HIDDEN)
    sval, oval = _tc_vals(pred_logits, W_sub, b_sub.reshape(1, HIDDEN),
                          W_obj, b_obj.reshape(1, HIDDEN),
                          ws_cls, wo_cls, b_cls.reshape(1, 1))
    mat_ref = jax.new_ref(jnp.zeros((NN,), jnp.float32))
    logits = _sc_scatter(pair_idx.reshape(P * 2 // 128, 128), sval, oval,
                         mat_ref)
    return logits, mat_ref[...].reshape(N_OBJ, N_OBJ)


# speedup vs baseline: 1.0051x; 1.0051x over previous
import functools

import jax
import jax.numpy as jnp
from jax import lax
from jax.experimental import pallas as pl
from jax.experimental.pallas import tpu as pltpu
from jax.experimental.pallas import tpu_sc as plsc

N_OBJ = 2048
NUM_CLS = 151
HIDDEN = 256
P = 131072
NN = N_OBJ * N_OBJ

NT = 16
PPT = P // NT
PROWS = PPT * 2 // 128
WROWS = 512
NPASS = N_OBJ // 2 // WROWS
WIN = WROWS * N_OBJ
TRASH = WIN
TSLICE = WIN // NT
ZB = 8192


def _tc_vals_body(lg_ref, ws_ref, bs_ref, wo_ref, bo_ref, wcs_ref, wco_ref,
                  bc_ref, sval_ref, oval_ref):
    x = lg_ref[...]
    m = jnp.max(x, axis=1, keepdims=True)
    e = jnp.exp(x - m)
    p = e / jnp.sum(e, axis=1, keepdims=True)
    hs = jnp.maximum(
        jnp.dot(p, ws_ref[...], preferred_element_type=jnp.float32) + bs_ref[...], 0.0)
    ho = jnp.maximum(
        jnp.dot(p, wo_ref[...], preferred_element_type=jnp.float32) + bo_ref[...], 0.0)
    sval_ref[...] = jnp.sum(hs * wcs_ref[...], axis=1) + bc_ref[0, 0]
    oval_ref[...] = jnp.sum(ho * wco_ref[...], axis=1)


_tc_vals = pl.pallas_call(
    _tc_vals_body,
    out_shape=(jax.ShapeDtypeStruct((N_OBJ,), jnp.float32),
               jax.ShapeDtypeStruct((N_OBJ,), jnp.float32)),
)


def _sc_body(pairs_hbm, sval_hbm, oval_hbm, logits_hbm, mat_hbm,
             pair_v, stab_v, otab_v, logit_v, score_v, fidx_v, pidx_v, zero_v,
             shared, stsem, ssem, lsem, zsem, osem):
    c = lax.axis_index("c")
    s = lax.axis_index("s")

    d1 = pltpu.async_copy(
        pairs_hbm.at[pl.ds(s * PROWS, PROWS), :], pair_v, stsem)
    d2 = pltpu.async_copy(sval_hbm, stab_v, stsem)
    d3 = pltpu.async_copy(oval_hbm, otab_v, stsem)

    def zfill(k, carry):
        zero_v[pl.ds(k * 16, 16)] = jnp.zeros((16,), jnp.float32)
        return carry
    lax.fori_loop(0, ZB // 16, zfill, 0)

    zcs = [
        pltpu.async_copy(
            zero_v, shared.at[pl.ds(s * TSLICE + z * ZB, ZB)], zsem)
        for z in range(TSLICE // ZB)
    ]

    d1.wait()
    d2.wait()
    d3.wait()

    def body(m, carry):
        lane = lax.iota(jnp.int32, 16)
        flat = m * 32 + 2 * lane
        ii = plsc.load_gather(pair_v, [flat // 128, flat % 128])
        jj = plsc.load_gather(pair_v, [(flat + 1) // 128, (flat + 1) % 128])
        sv = plsc.load_gather(stab_v, [ii])
        ov = plsc.load_gather(otab_v, [jj])
        lg = sv + ov
        logit_v[pl.ds(m * 16, 16)] = lg
        sc = 1.0 / (1.0 + jnp.exp(-lg))
        score_v[pl.ds(m * 16, 16)] = sc
        fidx_v[pl.ds(m * 16, 16)] = ii * N_OBJ + jj
        return carry
    lax.fori_loop(0, PPT // 16, body, 0)

    @pl.when(c == 0)
    def _():
        pltpu.async_copy(logit_v, logits_hbm.at[pl.ds(s * PPT, PPT)], lsem)

    for q in range(NPASS):
        pass_base = (c * (N_OBJ // 2) + q * WROWS) * N_OBJ

        def pbody(m, carry):
            fi = fidx_v[pl.ds(m * 16, 16)] - pass_base
            inw = (fi >= 0) & (fi < WIN)
            pidx_v[pl.ds(m * 16, 16)] = jnp.where(inw, fi, TRASH)
            return carry
        lax.fori_loop(0, PPT // 16, pbody, 0)

        for d in zcs:
            d.wait()
        plsc.subcore_barrier()
        pltpu.async_copy(score_v, shared.at[pidx_v], ssem).wait()
        plsc.subcore_barrier()

        pltpu.async_copy(
            shared.at[pl.ds(s * TSLICE, TSLICE)],
            mat_hbm.at[pl.ds(pass_base + s * TSLICE, TSLICE)], osem).wait()

        if q + 1 < NPASS:
            plsc.subcore_barrier()
            zcs = [
                pltpu.async_copy(
                    zero_v, shared.at[pl.ds(s * TSLICE + z * ZB, ZB)], zsem)
                for z in range(TSLICE // ZB)
            ]

    @pl.when(c == 0)
    def _():
        pltpu.make_async_copy(
            logit_v, logits_hbm.at[pl.ds(s * PPT, PPT)], lsem).wait()


_sc_scatter = functools.partial(
    pl.kernel,
    out_type=(jax.ShapeDtypeStruct((P,), jnp.float32),
              jax.ShapeDtypeStruct((NN,), jnp.float32)),
    mesh=plsc.VectorSubcoreMesh(core_axis_name="c", subcore_axis_name="s"),
    compiler_params=pltpu.CompilerParams(
        needs_layout_passes=False,
        disable_bounds_checks=True,
        disable_semaphore_checks=True,
    ),
    scratch_types=(
        pltpu.VMEM((PROWS, 128), jnp.int32),
        pltpu.VMEM((N_OBJ,), jnp.float32),
        pltpu.VMEM((N_OBJ,), jnp.float32),
        pltpu.VMEM((PPT,), jnp.float32),
        pltpu.VMEM((PPT,), jnp.float32),
        pltpu.VMEM((PPT,), jnp.int32),
        pltpu.VMEM((PPT,), jnp.int32),
        pltpu.VMEM((ZB,), jnp.float32),
        pltpu.VMEM_SHARED((WIN + 16,), jnp.float32),
        pltpu.SemaphoreType.DMA,
        pltpu.SemaphoreType.DMA,
        pltpu.SemaphoreType.DMA,
        pltpu.SemaphoreType.DMA,
        pltpu.SemaphoreType.DMA,
    ),
)(_sc_body)


def kernel(visual_feat, pred_logits, pair_idx, W_sub, b_sub, W_obj, b_obj,
           W_cls, b_cls):
    del visual_feat
    ws_cls = W_cls[:HIDDEN].reshape(1, HIDDEN)
    wo_cls = W_cls[HIDDEN:].reshape(1, HIDDEN)
    sval, oval = _tc_vals(pred_logits, W_sub, b_sub.reshape(1, HIDDEN),
                          W_obj, b_obj.reshape(1, HIDDEN),
                          ws_cls, wo_cls, b_cls.reshape(1, 1))
    logits, mat = _sc_scatter(pair_idx.reshape(P * 2 // 128, 128), sval, oval)
    return logits, mat.reshape(N_OBJ, N_OBJ)
